# baseline (speedup 1.0000x reference)
import functools

import jax
import jax.numpy as jnp
from jax import lax
from jax.experimental import pallas as pl
from jax.experimental.pallas import tpu as pltpu
from jax.experimental.pallas import tpu_sc as plsc

RADIUS = 1.8320508075688772
INV_RADIUS = 1.0 / RADIUS
N = 10000
E = 320000
F = 128
NC = 2
NS = 16
NW = NC * NS
EW = E // NW
CH_A = 2000
CH_C = 40
NPAD = 10240
ROWS_S = NPAD // NS
CNT_S = NPAD // NS

def _mesh():
    return plsc.VectorSubcoreMesh(core_axis_name="c", subcore_axis_name="s",
                                  num_cores=NC, num_subcores=NS)


CSUB = 125
NSUB = CH_A // CSUB


def _rel_body(pix, piy, piz, pox, poy, poz, src, dst, dst2, zeros1d, ones1d,
              relx, rely, relz, cnt_out,
              bix, biy, biz, box, boy, boz, sbuf, dbuf, rbx, rby, rbz,
              dbuf2, obuf, semc, cnt_sp):
    c = lax.axis_index("c")
    s = lax.axis_index("s")
    w = c * NS + s
    e0 = w * EW
    r0 = s * CNT_S
    pltpu.sync_copy(pix, bix)
    pltpu.sync_copy(piy, biy)
    pltpu.sync_copy(piz, biz)
    pltpu.sync_copy(pox, box)
    pltpu.sync_copy(poy, boy)
    pltpu.sync_copy(poz, boz)
    pltpu.sync_copy(zeros1d, cnt_sp.at[pl.ds(r0, CNT_S)])
    pltpu.sync_copy(ones1d, obuf)
    plsc.subcore_barrier()

    def chunk(k, _):
        eo = e0 + k * CH_A
        pltpu.sync_copy(src.at[pl.ds(eo, CH_A)], sbuf)
        pltpu.sync_copy(dst.at[pl.ds(eo, CH_A)], dbuf)
        pltpu.sync_copy(dst2.at[pl.ds(pl.multiple_of(eo // CSUB, 8), NSUB), :], dbuf2)
        for j in range(NSUB):
            pltpu.async_copy(obuf, cnt_sp.at[dbuf2.at[j]], semc, add=True)

        def group(g, _):
            sl = pl.ds(g * 16, 16)
            sv = sbuf[sl]
            dv = dbuf[sl]
            rbx[sl] = (plsc.load_gather(bix, [sv]) - plsc.load_gather(box, [dv])) * INV_RADIUS
            rby[sl] = (plsc.load_gather(biy, [sv]) - plsc.load_gather(boy, [dv])) * INV_RADIUS
            rbz[sl] = (plsc.load_gather(biz, [sv]) - plsc.load_gather(boz, [dv])) * INV_RADIUS
            return 0

        lax.fori_loop(0, CH_A // 16, group, 0)
        pltpu.sync_copy(rbx, relx.at[pl.ds(eo, CH_A)])
        pltpu.sync_copy(rby, rely.at[pl.ds(eo, CH_A)])
        pltpu.sync_copy(rbz, relz.at[pl.ds(eo, CH_A)])
        for j in range(NSUB):
            pltpu.make_async_copy(obuf, cnt_sp.at[dbuf2.at[j]], semc).wait()
        return 0

    lax.fori_loop(0, EW // CH_A, chunk, 0)
    plsc.subcore_barrier()
    pltpu.sync_copy(cnt_sp.at[pl.ds(r0, CNT_S)], cnt_out.at[c, pl.ds(r0, CNT_S)])


def _rel_kernel(*args):
  return pl.kernel(
    _rel_body,
    out_type=(
        jax.ShapeDtypeStruct((E,), jnp.float32),
        jax.ShapeDtypeStruct((E,), jnp.float32),
        jax.ShapeDtypeStruct((E,), jnp.float32),
        jax.ShapeDtypeStruct((NC, NPAD), jnp.float32),
    ),
    mesh=_mesh(),
    compiler_params=pltpu.CompilerParams(needs_layout_passes=False),
    scratch_types=[
        pltpu.VMEM((N,), jnp.float32),
        pltpu.VMEM((N,), jnp.float32),
        pltpu.VMEM((N,), jnp.float32),
        pltpu.VMEM((N,), jnp.float32),
        pltpu.VMEM((N,), jnp.float32),
        pltpu.VMEM((N,), jnp.float32),
        pltpu.VMEM((CH_A,), jnp.int32),
        pltpu.VMEM((CH_A,), jnp.int32),
        pltpu.VMEM((CH_A,), jnp.float32),
        pltpu.VMEM((CH_A,), jnp.float32),
        pltpu.VMEM((CH_A,), jnp.float32),
        pltpu.VMEM((NSUB, CSUB), jnp.int32),
        pltpu.VMEM((CSUB,), jnp.float32),
        pltpu.SemaphoreType.DMA,
        pltpu.VMEM_SHARED((NPAD,), jnp.float32),
    ],
  )(*args)


B_MLP = 3200


def _mlp_body(rel3, w1, b1, w2d, b2, w3d, b3, out):
    h1 = jnp.maximum(
        lax.dot_general(rel3[...], w1[...], (((0,), (0,)), ((), ())),
                        preferred_element_type=jnp.float32) + b1[...], 0.0)
    h2 = jnp.maximum(
        jnp.dot(h1.astype(jnp.bfloat16), w2d[...],
                preferred_element_type=jnp.float32) + b2[...], 0.0)
    kern = jnp.dot(h2.astype(jnp.bfloat16), w3d[...],
                   preferred_element_type=jnp.float32) + b3[...]
    lo = lax.bitcast_convert_type(
        kern[:, :F // 2].astype(jnp.bfloat16), jnp.uint16).astype(jnp.uint32)
    hi = lax.bitcast_convert_type(
        kern[:, F // 2:].astype(jnp.bfloat16), jnp.uint16).astype(jnp.uint32)
    out[...] = lax.bitcast_convert_type(lo | (hi << 16), jnp.int32)


def _mlp(rel3, w1, b1, w2d, b2, w3d, b3):
    grid = (E // B_MLP,)
    row = pl.BlockSpec((1, F), lambda i: (0, 0))
    mat = pl.BlockSpec((F, F), lambda i: (0, 0))
    return pl.pallas_call(
        _mlp_body,
        grid=grid,
        in_specs=[pl.BlockSpec((3, B_MLP), lambda i: (0, i)),
                  pl.BlockSpec((3, F), lambda i: (0, 0)),
                  row, mat, row, mat, row],
        out_specs=pl.BlockSpec((B_MLP, F // 2), lambda i: (i, 0)),
        out_shape=jax.ShapeDtypeStruct((E, F // 2), jnp.int32),
    )(rel3, w1, b1.reshape(1, F), w2d, b2.reshape(1, F), w3d, b3.reshape(1, F))


NCH = EW // CH_C
NPAIR = NCH // 2


def _agg_body(kern, src, dst, feat, zeros2d,
              acc_out,
              kbufa, kbufb, fbufa, fbufb, sbufa, sbufb, dbufa, dbufb,
              xbufa, xbufb,
              semia, semib, semga, semgb, semsa, semsb, acc_sp):
    c = lax.axis_index("c")
    s = lax.axis_index("s")
    w = c * NS + s
    e0 = w * EW
    r0 = s * ROWS_S
    pltpu.sync_copy(zeros2d, acc_sp.at[pl.ds(r0, ROWS_S), :])
    plsc.subcore_barrier()

    def start_in(k, sbuf, dbuf, kbuf, sem):
        eo = e0 + k * CH_C
        pltpu.async_copy(src.at[pl.ds(eo, CH_C)], sbuf, sem)
        pltpu.async_copy(dst.at[pl.ds(eo, CH_C)], dbuf, sem)
        pltpu.async_copy(kern.at[pl.ds(eo, CH_C), :], kbuf, sem)

    def wait_in(k, sbuf, dbuf, kbuf, sem):
        eo = e0 + k * CH_C
        pltpu.make_async_copy(src.at[pl.ds(eo, CH_C)], sbuf, sem).wait()
        pltpu.make_async_copy(dst.at[pl.ds(eo, CH_C)], dbuf, sem).wait()
        pltpu.make_async_copy(kern.at[pl.ds(eo, CH_C), :], kbuf, sem).wait()

    def mul_scatter(kbuf, fbuf, dbuf, xbuf, sem):
        def mul(i, _):
            for q in range(F // 32):
                v = kbuf[i, pl.ds(q * 16, 16)]
                klo = plsc.bitcast(v << 16, jnp.float32)
                khi = plsc.bitcast(v & jnp.int32(-65536), jnp.float32)
                sl0 = pl.ds(q * 16, 16)
                sl1 = pl.ds(F // 2 + q * 16, 16)
                fbuf[i, sl0] = fbuf[i, sl0] * klo
                fbuf[i, sl1] = fbuf[i, sl1] * khi
            return 0

        lax.fori_loop(0, CH_C, mul, 0)
        for off in (0, 16, CH_C - 16):
            xbuf[pl.ds(off, 16)] = dbuf[pl.ds(off, 16)]
        pltpu.async_copy(fbuf, acc_sp.at[xbuf], sem, add=True)

    start_in(0, sbufa, dbufa, kbufa, semia)
    start_in(1, sbufb, dbufb, kbufb, semib)

    def pair(p, _):
        ka = 2 * p
        kb = 2 * p + 1
        wait_in(ka, sbufa, dbufa, kbufa, semia)

        @pl.when(p > 0)
        def _():
            pltpu.make_async_copy(fbufa, acc_sp.at[xbufa], semsa).wait()

        pltpu.async_copy(feat.at[sbufa], fbufa, semga)
        wait_in(kb, sbufb, dbufb, kbufb, semib)

        @pl.when(p > 0)
        def _():
            pltpu.make_async_copy(fbufb, acc_sp.at[xbufb], semsb).wait()

        pltpu.async_copy(feat.at[sbufb], fbufb, semgb)
        pltpu.make_async_copy(feat.at[sbufa], fbufa, semga).wait()
        mul_scatter(kbufa, fbufa, dbufa, xbufa, semsa)

        @pl.when(p + 1 < NPAIR)
        def _():
            start_in(2 * p + 2, sbufa, dbufa, kbufa, semia)

        pltpu.make_async_copy(feat.at[sbufb], fbufb, semgb).wait()
        mul_scatter(kbufb, fbufb, dbufb, xbufb, semsb)

        @pl.when(p + 1 < NPAIR)
        def _():
            start_in(2 * p + 3, sbufb, dbufb, kbufb, semib)

        return 0

    lax.fori_loop(0, NPAIR, pair, 0)
    pltpu.make_async_copy(fbufa, acc_sp.at[xbufa], semsa).wait()
    pltpu.make_async_copy(fbufb, acc_sp.at[xbufb], semsb).wait()
    plsc.subcore_barrier()
    pltpu.sync_copy(acc_sp.at[pl.ds(r0, ROWS_S), :], acc_out.at[c, pl.ds(r0, ROWS_S), :])


def _agg_kernel(*args):
  return pl.kernel(
    _agg_body,
    out_type=jax.ShapeDtypeStruct((NC, NPAD, F), jnp.float32),
    mesh=_mesh(),
    compiler_params=pltpu.CompilerParams(needs_layout_passes=False),
    scratch_types=[
        pltpu.VMEM((CH_C, F // 2), jnp.int32),
        pltpu.VMEM((CH_C, F // 2), jnp.int32),
        pltpu.VMEM((CH_C, F), jnp.float32),
        pltpu.VMEM((CH_C, F), jnp.float32),
        pltpu.VMEM((CH_C,), jnp.int32),
        pltpu.VMEM((CH_C,), jnp.int32),
        pltpu.VMEM((CH_C,), jnp.int32),
        pltpu.VMEM((CH_C,), jnp.int32),
        pltpu.VMEM((CH_C,), jnp.int32),
        pltpu.VMEM((CH_C,), jnp.int32),
        pltpu.SemaphoreType.DMA,
        pltpu.SemaphoreType.DMA,
        pltpu.SemaphoreType.DMA,
        pltpu.SemaphoreType.DMA,
        pltpu.SemaphoreType.DMA,
        pltpu.SemaphoreType.DMA,
        pltpu.VMEM_SHARED((NPAD, F), jnp.float32),
    ],
  )(*args)


R_DIV = 2000


def _div_body(acc, cnt, out):
    inv = 1.0 / jnp.maximum(cnt[0] + cnt[1], 1.0)
    out[...] = (acc[0] + acc[1]) * inv


def _divide(acc, cnt):
    grid = (N // R_DIV,)
    return pl.pallas_call(
        _div_body,
        grid=grid,
        in_specs=[
            pl.BlockSpec((NC, R_DIV, F), lambda i: (0, i, 0)),
            pl.BlockSpec((NC, R_DIV, 1), lambda i: (0, i, 0)),
        ],
        out_specs=pl.BlockSpec((R_DIV, F), lambda i: (i, 0)),
        out_shape=jax.ShapeDtypeStruct((N, F), jnp.float32),
    )(acc, cnt)


def _block_diag(w):
    bs = w.shape[0]
    nb = w.shape[1] // bs
    blocks = w.reshape(bs, nb, bs).transpose(1, 0, 2)
    eye = jnp.eye(nb, dtype=w.dtype)
    d = jnp.einsum('nbc,nm->nbmc', blocks, eye)
    return d.reshape(nb * bs, nb * bs)


def kernel(inFeatures, inPositions, outPositions, edge_index,
           weights, biases, weights2, biases2, weights3, biases3):
    src = edge_index[0].astype(jnp.int32)
    dst = edge_index[1].astype(jnp.int32)
    pix, piy, piz = (inPositions[:, 0], inPositions[:, 1], inPositions[:, 2])
    pox, poy, poz = (outPositions[:, 0], outPositions[:, 1], outPositions[:, 2])

    dst2 = dst.reshape(E // CSUB, CSUB)
    zeros1d = jnp.zeros((CNT_S,), jnp.float32)
    ones1d = jnp.ones((CSUB,), jnp.float32)
    relx, rely, relz, cnt = _rel_kernel(pix, piy, piz, pox, poy, poz, src, dst,
                                        dst2, zeros1d, ones1d)

    w2d = _block_diag(weights2).astype(jnp.bfloat16)
    w3d = _block_diag(weights3).astype(jnp.bfloat16)
    rel3 = jnp.stack([relx, rely, relz])
    kern = _mlp(rel3, weights, biases, w2d, biases2, w3d, biases3)

    zeros2d = jnp.zeros((ROWS_S, F), jnp.float32)
    acc = _agg_kernel(kern, src, dst, inFeatures, zeros2d)

    cnt = cnt[:, :N].reshape(NC, N, 1)
    return _divide(acc[:, :N, :], cnt)


# speedup vs baseline: 1.0310x; 1.0310x over previous
import functools

import jax
import jax.numpy as jnp
from jax import lax
from jax.experimental import pallas as pl
from jax.experimental.pallas import tpu as pltpu
from jax.experimental.pallas import tpu_sc as plsc

RADIUS = 1.8320508075688772
INV_RADIUS = 1.0 / RADIUS
N = 10000
E = 320000
F = 128
NC = 2
NS = 16
NW = NC * NS
EW = E // NW
CH_A = 2000
CH_C = 40
NPAD = 10240
ROWS_S = NPAD // NS
CNT_S = NPAD // NS

def _mesh():
    return plsc.VectorSubcoreMesh(core_axis_name="c", subcore_axis_name="s",
                                  num_cores=NC, num_subcores=NS)


CSUB = 125
NSUB = CH_A // CSUB


def _rel_body(pix, piy, piz, pox, poy, poz, src, dst, dst2, zeros1d, ones1d,
              relx, rely, relz, cnt_out,
              bix, biy, biz, box, boy, boz, sbuf, dbuf, rbx, rby, rbz,
              dbuf2, obuf, semc, cnt_sp):
    c = lax.axis_index("c")
    s = lax.axis_index("s")
    w = c * NS + s
    e0 = w * EW
    r0 = s * CNT_S
    pltpu.sync_copy(pix, bix)
    pltpu.sync_copy(piy, biy)
    pltpu.sync_copy(piz, biz)
    pltpu.sync_copy(pox, box)
    pltpu.sync_copy(poy, boy)
    pltpu.sync_copy(poz, boz)
    pltpu.sync_copy(zeros1d, cnt_sp.at[pl.ds(r0, CNT_S)])
    pltpu.sync_copy(ones1d, obuf)
    plsc.subcore_barrier()

    def chunk(k, _):
        eo = e0 + k * CH_A
        pltpu.sync_copy(src.at[pl.ds(eo, CH_A)], sbuf)
        pltpu.sync_copy(dst.at[pl.ds(eo, CH_A)], dbuf)
        pltpu.sync_copy(dst2.at[pl.ds(pl.multiple_of(eo // CSUB, 8), NSUB), :], dbuf2)
        for j in range(NSUB):
            pltpu.async_copy(obuf, cnt_sp.at[dbuf2.at[j]], semc, add=True)

        def group(g, _):
            sl = pl.ds(g * 16, 16)
            sv = sbuf[sl]
            dv = dbuf[sl]
            rbx[sl] = (plsc.load_gather(bix, [sv]) - plsc.load_gather(box, [dv])) * INV_RADIUS
            rby[sl] = (plsc.load_gather(biy, [sv]) - plsc.load_gather(boy, [dv])) * INV_RADIUS
            rbz[sl] = (plsc.load_gather(biz, [sv]) - plsc.load_gather(boz, [dv])) * INV_RADIUS
            return 0

        lax.fori_loop(0, CH_A // 16, group, 0)
        pltpu.sync_copy(rbx, relx.at[pl.ds(eo, CH_A)])
        pltpu.sync_copy(rby, rely.at[pl.ds(eo, CH_A)])
        pltpu.sync_copy(rbz, relz.at[pl.ds(eo, CH_A)])
        for j in range(NSUB):
            pltpu.make_async_copy(obuf, cnt_sp.at[dbuf2.at[j]], semc).wait()
        return 0

    lax.fori_loop(0, EW // CH_A, chunk, 0)
    plsc.subcore_barrier()
    pltpu.sync_copy(cnt_sp.at[pl.ds(r0, CNT_S)], cnt_out.at[c, pl.ds(r0, CNT_S)])


def _rel_kernel(*args):
  return pl.kernel(
    _rel_body,
    out_type=(
        jax.ShapeDtypeStruct((E,), jnp.float32),
        jax.ShapeDtypeStruct((E,), jnp.float32),
        jax.ShapeDtypeStruct((E,), jnp.float32),
        jax.ShapeDtypeStruct((NC, NPAD), jnp.float32),
    ),
    mesh=_mesh(),
    compiler_params=pltpu.CompilerParams(needs_layout_passes=False),
    scratch_types=[
        pltpu.VMEM((N,), jnp.float32),
        pltpu.VMEM((N,), jnp.float32),
        pltpu.VMEM((N,), jnp.float32),
        pltpu.VMEM((N,), jnp.float32),
        pltpu.VMEM((N,), jnp.float32),
        pltpu.VMEM((N,), jnp.float32),
        pltpu.VMEM((CH_A,), jnp.int32),
        pltpu.VMEM((CH_A,), jnp.int32),
        pltpu.VMEM((CH_A,), jnp.float32),
        pltpu.VMEM((CH_A,), jnp.float32),
        pltpu.VMEM((CH_A,), jnp.float32),
        pltpu.VMEM((NSUB, CSUB), jnp.int32),
        pltpu.VMEM((CSUB,), jnp.float32),
        pltpu.SemaphoreType.DMA,
        pltpu.VMEM_SHARED((NPAD,), jnp.float32),
    ],
  )(*args)


B_MLP = 6400


def _mlp_body(rel3, w1, b1, w2d, b2, w3d, b3, out):
    h1 = jnp.maximum(
        lax.dot_general(rel3[...], w1[...], (((0,), (0,)), ((), ())),
                        preferred_element_type=jnp.float32) + b1[...], 0.0)
    h2 = jnp.maximum(
        jnp.dot(h1.astype(jnp.bfloat16), w2d[...],
                preferred_element_type=jnp.float32) + b2[...], 0.0)
    kern = jnp.dot(h2.astype(jnp.bfloat16), w3d[...],
                   preferred_element_type=jnp.float32) + b3[...]
    lo = lax.bitcast_convert_type(
        kern[:, :F // 2].astype(jnp.bfloat16), jnp.uint16).astype(jnp.uint32)
    hi = lax.bitcast_convert_type(
        kern[:, F // 2:].astype(jnp.bfloat16), jnp.uint16).astype(jnp.uint32)
    out[...] = lax.bitcast_convert_type(lo | (hi << 16), jnp.int32)


def _mlp(rel3, w1, b1, w2d, b2, w3d, b3):
    grid = (E // B_MLP,)
    row = pl.BlockSpec((1, F), lambda i: (0, 0))
    mat = pl.BlockSpec((F, F), lambda i: (0, 0))
    return pl.pallas_call(
        _mlp_body,
        grid=grid,
        in_specs=[pl.BlockSpec((3, B_MLP), lambda i: (0, i)),
                  pl.BlockSpec((3, F), lambda i: (0, 0)),
                  row, mat, row, mat, row],
        out_specs=pl.BlockSpec((B_MLP, F // 2), lambda i: (i, 0)),
        out_shape=jax.ShapeDtypeStruct((E, F // 2), jnp.int32),
    )(rel3, w1, b1.reshape(1, F), w2d, b2.reshape(1, F), w3d, b3.reshape(1, F))


NCH = EW // CH_C
NPAIR = NCH // 2


def _agg_body(kern, src, dst, feat, zeros2d,
              acc_out,
              kbufa, kbufb, fbufa, fbufb, sbufa, sbufb, dbufa, dbufb,
              xbufa, xbufb,
              semia, semib, semga, semgb, semsa, semsb, acc_sp):
    c = lax.axis_index("c")
    s = lax.axis_index("s")
    w = c * NS + s
    e0 = w * EW
    r0 = s * ROWS_S
    pltpu.sync_copy(zeros2d, acc_sp.at[pl.ds(r0, ROWS_S), :])
    plsc.subcore_barrier()

    def start_in(k, sbuf, dbuf, kbuf, sem):
        eo = e0 + k * CH_C
        pltpu.async_copy(src.at[pl.ds(eo, CH_C)], sbuf, sem)
        pltpu.async_copy(dst.at[pl.ds(eo, CH_C)], dbuf, sem)
        pltpu.async_copy(kern.at[pl.ds(eo, CH_C), :], kbuf, sem)

    def wait_in(k, sbuf, dbuf, kbuf, sem):
        eo = e0 + k * CH_C
        pltpu.make_async_copy(src.at[pl.ds(eo, CH_C)], sbuf, sem).wait()
        pltpu.make_async_copy(dst.at[pl.ds(eo, CH_C)], dbuf, sem).wait()
        pltpu.make_async_copy(kern.at[pl.ds(eo, CH_C), :], kbuf, sem).wait()

    def mul_scatter(kbuf, fbuf, dbuf, xbuf, sem):
        def mul(i, _):
            for q in range(F // 32):
                v = kbuf[i, pl.ds(q * 16, 16)]
                klo = plsc.bitcast(v << 16, jnp.float32)
                khi = plsc.bitcast(v & jnp.int32(-65536), jnp.float32)
                sl0 = pl.ds(q * 16, 16)
                sl1 = pl.ds(F // 2 + q * 16, 16)
                fbuf[i, sl0] = fbuf[i, sl0] * klo
                fbuf[i, sl1] = fbuf[i, sl1] * khi
            return 0

        lax.fori_loop(0, CH_C, mul, 0)
        for off in (0, 16, CH_C - 16):
            xbuf[pl.ds(off, 16)] = dbuf[pl.ds(off, 16)]
        pltpu.async_copy(fbuf, acc_sp.at[xbuf], sem, add=True)

    start_in(0, sbufa, dbufa, kbufa, semia)
    start_in(1, sbufb, dbufb, kbufb, semib)

    def pair(p, _):
        ka = 2 * p
        kb = 2 * p + 1
        wait_in(ka, sbufa, dbufa, kbufa, semia)

        @pl.when(p > 0)
        def _():
            pltpu.make_async_copy(fbufa, acc_sp.at[xbufa], semsa).wait()

        pltpu.async_copy(feat.at[sbufa], fbufa, semga)
        wait_in(kb, sbufb, dbufb, kbufb, semib)

        @pl.when(p > 0)
        def _():
            pltpu.make_async_copy(fbufb, acc_sp.at[xbufb], semsb).wait()

        pltpu.async_copy(feat.at[sbufb], fbufb, semgb)
        pltpu.make_async_copy(feat.at[sbufa], fbufa, semga).wait()
        mul_scatter(kbufa, fbufa, dbufa, xbufa, semsa)

        @pl.when(p + 1 < NPAIR)
        def _():
            start_in(2 * p + 2, sbufa, dbufa, kbufa, semia)

        pltpu.make_async_copy(feat.at[sbufb], fbufb, semgb).wait()
        mul_scatter(kbufb, fbufb, dbufb, xbufb, semsb)

        @pl.when(p + 1 < NPAIR)
        def _():
            start_in(2 * p + 3, sbufb, dbufb, kbufb, semib)

        return 0

    lax.fori_loop(0, NPAIR, pair, 0)
    pltpu.make_async_copy(fbufa, acc_sp.at[xbufa], semsa).wait()
    pltpu.make_async_copy(fbufb, acc_sp.at[xbufb], semsb).wait()
    plsc.subcore_barrier()
    pltpu.sync_copy(acc_sp.at[pl.ds(r0, ROWS_S), :], acc_out.at[c, pl.ds(r0, ROWS_S), :])


def _agg_kernel(*args):
  return pl.kernel(
    _agg_body,
    out_type=jax.ShapeDtypeStruct((NC, NPAD, F), jnp.float32),
    mesh=_mesh(),
    compiler_params=pltpu.CompilerParams(needs_layout_passes=False),
    scratch_types=[
        pltpu.VMEM((CH_C, F // 2), jnp.int32),
        pltpu.VMEM((CH_C, F // 2), jnp.int32),
        pltpu.VMEM((CH_C, F), jnp.float32),
        pltpu.VMEM((CH_C, F), jnp.float32),
        pltpu.VMEM((CH_C,), jnp.int32),
        pltpu.VMEM((CH_C,), jnp.int32),
        pltpu.VMEM((CH_C,), jnp.int32),
        pltpu.VMEM((CH_C,), jnp.int32),
        pltpu.VMEM((CH_C,), jnp.int32),
        pltpu.VMEM((CH_C,), jnp.int32),
        pltpu.SemaphoreType.DMA,
        pltpu.SemaphoreType.DMA,
        pltpu.SemaphoreType.DMA,
        pltpu.SemaphoreType.DMA,
        pltpu.SemaphoreType.DMA,
        pltpu.SemaphoreType.DMA,
        pltpu.VMEM_SHARED((NPAD, F), jnp.float32),
    ],
  )(*args)


R_DIV = 2000


def _div_body(acc, cnt, out):
    inv = 1.0 / jnp.maximum(cnt[0] + cnt[1], 1.0)
    out[...] = (acc[0] + acc[1]) * inv


def _divide(acc, cnt):
    grid = (N // R_DIV,)
    return pl.pallas_call(
        _div_body,
        grid=grid,
        in_specs=[
            pl.BlockSpec((NC, R_DIV, F), lambda i: (0, i, 0)),
            pl.BlockSpec((NC, R_DIV, 1), lambda i: (0, i, 0)),
        ],
        out_specs=pl.BlockSpec((R_DIV, F), lambda i: (i, 0)),
        out_shape=jax.ShapeDtypeStruct((N, F), jnp.float32),
    )(acc, cnt)


def _block_diag(w):
    bs = w.shape[0]
    nb = w.shape[1] // bs
    blocks = w.reshape(bs, nb, bs).transpose(1, 0, 2)
    eye = jnp.eye(nb, dtype=w.dtype)
    d = jnp.einsum('nbc,nm->nbmc', blocks, eye)
    return d.reshape(nb * bs, nb * bs)


def kernel(inFeatures, inPositions, outPositions, edge_index,
           weights, biases, weights2, biases2, weights3, biases3):
    src = edge_index[0].astype(jnp.int32)
    dst = edge_index[1].astype(jnp.int32)
    pix, piy, piz = (inPositions[:, 0], inPositions[:, 1], inPositions[:, 2])
    pox, poy, poz = (outPositions[:, 0], outPositions[:, 1], outPositions[:, 2])

    dst2 = dst.reshape(E // CSUB, CSUB)
    zeros1d = jnp.zeros((CNT_S,), jnp.float32)
    ones1d = jnp.ones((CSUB,), jnp.float32)
    relx, rely, relz, cnt = _rel_kernel(pix, piy, piz, pox, poy, poz, src, dst,
                                        dst2, zeros1d, ones1d)

    w2d = _block_diag(weights2).astype(jnp.bfloat16)
    w3d = _block_diag(weights3).astype(jnp.bfloat16)
    rel3 = jnp.stack([relx, rely, relz])
    kern = _mlp(rel3, weights, biases, w2d, biases2, w3d, biases3)

    zeros2d = jnp.zeros((ROWS_S, F), jnp.float32)
    acc = _agg_kernel(kern, src, dst, inFeatures, zeros2d)

    cnt = cnt[:, :N].reshape(NC, N, 1)
    return _divide(acc[:, :N, :], cnt)
